# baseline (speedup 1.0000x reference)
import jax
import jax.numpy as jnp
from jax.experimental import pallas as pl

SHAPE = (16, 32, 32)
D_PER = 80
EMBD = 240


TB = 8


def _tile_kernel(w0_ref, w1_ref, w2_ref, out_ref):
    j = pl.program_id(1)
    T, H, W = SHAPE
    w0 = w0_ref[pl.ds(j * TB, TB), :]
    a = jnp.broadcast_to(w0[:, None, None, :], (TB, H, W, D_PER))
    b = jnp.broadcast_to(w1_ref[...][None, :, None, :], (TB, H, W, D_PER))
    c = jnp.broadcast_to(w2_ref[...][None, None, :, :], (TB, H, W, D_PER))
    out_ref[0] = jnp.concatenate([a, b, c], axis=-1)


def kernel(x, W0, W1, W2):
    B = x.shape[0]
    T, H, W = SHAPE
    grid = (B, T // TB)
    return pl.pallas_call(
        _tile_kernel,
        grid=grid,
        in_specs=[
            pl.BlockSpec((T, D_PER), lambda b, t: (0, 0)),
            pl.BlockSpec((H, D_PER), lambda b, t: (0, 0)),
            pl.BlockSpec((W, D_PER), lambda b, t: (0, 0)),
        ],
        out_specs=pl.BlockSpec(
            (1, TB, H, W, EMBD), lambda b, t: (b, t, 0, 0, 0)
        ),
        out_shape=jax.ShapeDtypeStruct((B, T, H, W, EMBD), jnp.float32),
    )(W0, W1, W2)


# speedup vs baseline: 1.0455x; 1.0455x over previous
import jax
import jax.numpy as jnp
from jax.experimental import pallas as pl
import jax.experimental.pallas.tpu as pltpu

SHAPE = (16, 32, 32)
D_PER = 80
EMBD = 240
TB = 4


def _build_kernel(w0_ref, w1_ref, w2_ref, out_ref, scratch, sems):
    T, H, W = SHAPE
    B = out_ref.shape[0]
    n_chunks = T // TB
    w1b = jnp.broadcast_to(w1_ref[...][None, :, None, :], (TB, H, W, D_PER))
    w2b = jnp.broadcast_to(w2_ref[...][None, None, :, :], (TB, H, W, D_PER))
    for j in range(n_chunks):
        w0 = w0_ref[pl.ds(j * TB, TB), :]
        a = jnp.broadcast_to(w0[:, None, None, :], (TB, H, W, D_PER))
        scratch[pl.ds(j * TB, TB)] = jnp.concatenate([a, w1b, w2b], axis=-1)
        for b in range(B):
            pltpu.make_async_copy(
                scratch.at[pl.ds(j * TB, TB)],
                out_ref.at[b, pl.ds(j * TB, TB)],
                sems.at[b],
            ).start()
    for j in range(n_chunks):
        for b in range(B):
            pltpu.make_async_copy(
                scratch.at[pl.ds(j * TB, TB)],
                out_ref.at[b, pl.ds(j * TB, TB)],
                sems.at[b],
            ).wait()


def kernel(x, W0, W1, W2):
    B = x.shape[0]
    T, H, W = SHAPE
    return pl.pallas_call(
        _build_kernel,
        in_specs=[
            pl.BlockSpec(memory_space=pltpu.MemorySpace.VMEM),
            pl.BlockSpec(memory_space=pltpu.MemorySpace.VMEM),
            pl.BlockSpec(memory_space=pltpu.MemorySpace.VMEM),
        ],
        out_specs=pl.BlockSpec(memory_space=pl.ANY),
        out_shape=jax.ShapeDtypeStruct((B, T, H, W, EMBD), jnp.float32),
        scratch_shapes=[
            pltpu.MemorySpace.VMEM((T, H, W, EMBD), jnp.float32),
            pltpu.SemaphoreType.DMA((B,)),
        ],
    )(W0, W1, W2)
